# baseline (speedup 1.0000x reference)
import functools

import jax
import jax.numpy as jnp
from jax import lax
from jax.experimental import pallas as pl
from jax.experimental.pallas import tpu as pltpu
from jax.experimental.pallas import tpu_sc as plsc

N_NODES = 50000
D_IN = 128
HIDDEN = 256
OUT = 128
NUM_GRAPHS = 256

NC = 2
NS = 16
NW = NC * NS
BLK = 80
NBLK = N_NODES // BLK
BASE_BLK = NBLK // NW
EXTRA = NBLK - BASE_BLK * NW
CW = D_IN + 16
NVEC = D_IN // 16 + 1


@functools.partial(
    pl.kernel,
    out_type=jax.ShapeDtypeStruct((NW, NUM_GRAPHS, CW), jnp.float32),
    mesh=plsc.VectorSubcoreMesh(core_axis_name="c", subcore_axis_name="s"),
    scratch_types=[
        pltpu.VMEM((6 * BLK, D_IN), jnp.float32),
        pltpu.VMEM((6, BLK + 16), jnp.int32),
        pltpu.VMEM((NUM_GRAPHS + 1, CW), jnp.float32),
        pltpu.SemaphoreType.DMA,
        pltpu.SemaphoreType.DMA,
    ],
)
def _sc_segsum(x_hbm, ids_hbm, out_hbm, xbuf, idbuf, acc, semx, semi):
    c = lax.axis_index("c")
    s = lax.axis_index("s")
    w = c * NS + s

    iota = lax.iota(jnp.int32, 16)
    cntv = jnp.where(iota == 0, 1.0, 0.0).astype(jnp.float32)
    cnt16v = jnp.where(iota == 0, 16.0, 0.0).astype(jnp.float32)
    zeros16 = jnp.zeros((16,), jnp.float32)

    def zero_body(i, carry):
        for g in range(CW // 16):
            acc[i, pl.ds(g * 16, 16)] = zeros16
        return carry

    lax.fori_loop(0, NUM_GRAPHS + 1, zero_body, 0, unroll=4)

    start = w * BASE_BLK + jnp.minimum(w, EXTRA)
    nblk = jnp.where(w < EXTRA, BASE_BLK + 1, BASE_BLK)

    def issue(i, par):
        blk = start + i
        pltpu.async_copy(
            x_hbm.at[pl.ds(blk * BLK, BLK), :],
            xbuf.at[pl.ds(par * BLK, BLK), :], semx)
        pltpu.async_copy(
            ids_hbm.at[pl.ds(blk * BLK, BLK)], idbuf.at[par, pl.ds(0, BLK)],
            semi)

    def drain(i, par):
        blk = start + i
        pltpu.make_async_copy(
            x_hbm.at[pl.ds(blk * BLK, BLK), :],
            xbuf.at[pl.ds(par * BLK, BLK), :], semx).wait()
        pltpu.make_async_copy(
            ids_hbm.at[pl.ds(blk * BLK, BLK)], idbuf.at[par, pl.ds(0, BLK)],
            semi).wait()

    for j in range(5):
        issue(j, j)

    def blk_body(i, carry):
        par = lax.rem(i, 6)
        drain(i, par)

        @pl.when(i + 5 < nblk)
        def _():
            issue(i + 5, lax.rem(i + 5, 6))

        pb = par * BLK

        def grp_body(gidx, gc):
            r0 = pb + gidx * 16
            idvec = idbuf[par, pl.ds(gidx * 16, 16)]
            seg0 = idvec[0]
            last = idvec[15]
            uniform = seg0 == last

            gs = [xbuf[r0, pl.ds(g * 16, 16)] for g in range(D_IN // 16)]
            for r in range(1, 16):
                for g in range(D_IN // 16):
                    gs[g] = gs[g] + xbuf[r0 + r, pl.ds(g * 16, 16)]

            @pl.when(uniform)
            def _():
                for g in range(D_IN // 16):
                    acc[seg0, pl.ds(g * 16, 16)] = (
                        acc[seg0, pl.ds(g * 16, 16)] + gs[g])
                acc[seg0, pl.ds(D_IN, 16)] = acc[seg0, pl.ds(D_IN, 16)] + cnt16v

            @pl.when(jnp.logical_not(uniform))
            def _():
                for r in range(16):
                    seg_r = idvec[r]
                    for g in range(D_IN // 16):
                        acc[seg_r, pl.ds(g * 16, 16)] = (
                            acc[seg_r, pl.ds(g * 16, 16)]
                            + xbuf[r0 + r, pl.ds(g * 16, 16)])
                    acc[seg_r, pl.ds(D_IN, 16)] = (
                        acc[seg_r, pl.ds(D_IN, 16)] + cntv)

            return gc

        return lax.fori_loop(0, BLK // 16, grp_body, carry)

    lax.fori_loop(0, nblk, blk_body, 0)

    pltpu.sync_copy(acc.at[pl.ds(0, NUM_GRAPHS), :], out_hbm.at[w])


def _tc_body(sacc_ref, w_in_ref, b_in_ref, w_pred_ref, b_pred_ref, out_ref):
    a = jnp.sum(sacc_ref[...], axis=0)
    sx = a[:, :D_IN]
    ext = a[:, D_IN:]
    row0 = lax.broadcasted_iota(jnp.int32, (16, HIDDEN), 0) == 0
    b16 = jnp.where(row0, jnp.broadcast_to(b_in_ref[...], (16, HIDDEN)), 0.0)
    pooled = jnp.dot(sx, w_in_ref[...], preferred_element_type=jnp.float32)
    pooled = pooled + jnp.dot(ext, b16, preferred_element_type=jnp.float32)
    out_ref[...] = (
        jnp.dot(pooled, w_pred_ref[...], preferred_element_type=jnp.float32)
        + b_pred_ref[...]
    )


def kernel(X, params, graph_segment_ids, W_in, b_in, W_pred, b_pred):
    del params
    ids32 = graph_segment_ids.astype(jnp.int32)
    sacc = _sc_segsum(X, ids32)
    out = pl.pallas_call(
        _tc_body,
        out_shape=jax.ShapeDtypeStruct((NUM_GRAPHS, OUT), jnp.float32),
    )(sacc, W_in, b_in.reshape(1, HIDDEN), W_pred, b_pred.reshape(1, OUT))
    return out


# speedup vs baseline: 1.0012x; 1.0012x over previous
import functools

import jax
import jax.numpy as jnp
from jax import lax
from jax.experimental import pallas as pl
from jax.experimental.pallas import tpu as pltpu
from jax.experimental.pallas import tpu_sc as plsc

N_NODES = 50000
D_IN = 128
HIDDEN = 256
OUT = 128
NUM_GRAPHS = 256

NC = 2
NS = 16
NW = NC * NS
BLK = 80
NBLK = N_NODES // BLK
BASE_BLK = NBLK // NW
EXTRA = NBLK - BASE_BLK * NW
CW = D_IN + 16
NVEC = D_IN // 16 + 1


@functools.partial(
    pl.kernel,
    out_type=jax.ShapeDtypeStruct((NW, NUM_GRAPHS, CW), jnp.float32),
    mesh=plsc.VectorSubcoreMesh(core_axis_name="c", subcore_axis_name="s"),
    scratch_types=[
        pltpu.VMEM((4 * BLK, D_IN), jnp.float32),
        pltpu.VMEM((4, BLK + 16), jnp.int32),
        pltpu.VMEM((NUM_GRAPHS + 1, CW), jnp.float32),
        pltpu.SemaphoreType.DMA,
        pltpu.SemaphoreType.DMA,
    ],
)
def _sc_segsum(x_hbm, ids_hbm, out_hbm, xbuf, idbuf, acc, semx, semi):
    c = lax.axis_index("c")
    s = lax.axis_index("s")
    w = c * NS + s

    iota = lax.iota(jnp.int32, 16)
    cntv = jnp.where(iota == 0, 1.0, 0.0).astype(jnp.float32)
    cnt16v = jnp.where(iota == 0, 16.0, 0.0).astype(jnp.float32)
    zeros16 = jnp.zeros((16,), jnp.float32)

    def zero_body(i, carry):
        for g in range(CW // 16):
            acc[i, pl.ds(g * 16, 16)] = zeros16
        return carry

    lax.fori_loop(0, NUM_GRAPHS + 1, zero_body, 0, unroll=4)

    start = w * BASE_BLK + jnp.minimum(w, EXTRA)
    nblk = jnp.where(w < EXTRA, BASE_BLK + 1, BASE_BLK)

    def issue(i, par):
        blk = start + i
        pltpu.async_copy(
            x_hbm.at[pl.ds(blk * BLK, BLK), :],
            xbuf.at[pl.ds(par * BLK, BLK), :], semx)
        pltpu.async_copy(
            ids_hbm.at[pl.ds(blk * BLK, BLK)], idbuf.at[par, pl.ds(0, BLK)],
            semi)

    def drain(i, par):
        blk = start + i
        pltpu.make_async_copy(
            x_hbm.at[pl.ds(blk * BLK, BLK), :],
            xbuf.at[pl.ds(par * BLK, BLK), :], semx).wait()
        pltpu.make_async_copy(
            ids_hbm.at[pl.ds(blk * BLK, BLK)], idbuf.at[par, pl.ds(0, BLK)],
            semi).wait()

    for j in range(3):
        issue(j, j)

    def blk_body(i, carry):
        par = jnp.bitwise_and(i, 3)
        drain(i, par)

        @pl.when(i + 3 < nblk)
        def _():
            issue(i + 3, jnp.bitwise_and(i + 3, 3))

        pb = par * BLK

        def grp_body(gidx, gc):
            r0 = pb + gidx * 16
            idvec = idbuf[par, pl.ds(gidx * 16, 16)]
            seg0 = idvec[0]
            last = idvec[15]
            uniform = seg0 == last

            gs = [xbuf[r0, pl.ds(g * 16, 16)] for g in range(D_IN // 16)]
            for r in range(1, 16):
                for g in range(D_IN // 16):
                    gs[g] = gs[g] + xbuf[r0 + r, pl.ds(g * 16, 16)]

            @pl.when(uniform)
            def _():
                for g in range(D_IN // 16):
                    acc[seg0, pl.ds(g * 16, 16)] = (
                        acc[seg0, pl.ds(g * 16, 16)] + gs[g])
                acc[seg0, pl.ds(D_IN, 16)] = acc[seg0, pl.ds(D_IN, 16)] + cnt16v

            @pl.when(jnp.logical_not(uniform))
            def _():
                for r in range(16):
                    seg_r = idvec[r]
                    for g in range(D_IN // 16):
                        acc[seg_r, pl.ds(g * 16, 16)] = (
                            acc[seg_r, pl.ds(g * 16, 16)]
                            + xbuf[r0 + r, pl.ds(g * 16, 16)])
                    acc[seg_r, pl.ds(D_IN, 16)] = (
                        acc[seg_r, pl.ds(D_IN, 16)] + cntv)

            return gc

        return lax.fori_loop(0, BLK // 16, grp_body, carry)

    lax.fori_loop(0, nblk, blk_body, 0)

    pltpu.sync_copy(acc.at[pl.ds(0, NUM_GRAPHS), :], out_hbm.at[w])


def _tc_body(sacc_ref, w_in_ref, b_in_ref, w_pred_ref, b_pred_ref, out_ref):
    a = jnp.sum(sacc_ref[...], axis=0)
    sx = a[:, :D_IN]
    ext = a[:, D_IN:]
    row0 = lax.broadcasted_iota(jnp.int32, (16, HIDDEN), 0) == 0
    b16 = jnp.where(row0, jnp.broadcast_to(b_in_ref[...], (16, HIDDEN)), 0.0)
    pooled = jnp.dot(sx, w_in_ref[...], preferred_element_type=jnp.float32)
    pooled = pooled + jnp.dot(ext, b16, preferred_element_type=jnp.float32)
    out_ref[...] = (
        jnp.dot(pooled, w_pred_ref[...], preferred_element_type=jnp.float32)
        + b_pred_ref[...]
    )


def kernel(X, params, graph_segment_ids, W_in, b_in, W_pred, b_pred):
    del params
    ids32 = graph_segment_ids.astype(jnp.int32)
    sacc = _sc_segsum(X, ids32)
    out = pl.pallas_call(
        _tc_body,
        out_shape=jax.ShapeDtypeStruct((NUM_GRAPHS, OUT), jnp.float32),
    )(sacc, W_in, b_in.reshape(1, HIDDEN), W_pred, b_pred.reshape(1, OUT))
    return out


# speedup vs baseline: 1.0017x; 1.0005x over previous
import functools

import jax
import jax.numpy as jnp
from jax import lax
from jax.experimental import pallas as pl
from jax.experimental.pallas import tpu as pltpu
from jax.experimental.pallas import tpu_sc as plsc

N_NODES = 50000
D_IN = 128
HIDDEN = 256
OUT = 128
NUM_GRAPHS = 256

NC = 2
NS = 16
NW = NC * NS
BLK = 80
NBLK = N_NODES // BLK
BASE_BLK = NBLK // NW
EXTRA = NBLK - BASE_BLK * NW
CW = D_IN + 16


@functools.partial(
    pl.kernel,
    out_type=jax.ShapeDtypeStruct((NW, NUM_GRAPHS, CW), jnp.float32),
    mesh=plsc.VectorSubcoreMesh(core_axis_name="c", subcore_axis_name="s"),
    scratch_types=[
        pltpu.VMEM((4 * BLK, D_IN), jnp.float32),
        pltpu.VMEM((4, BLK + 16), jnp.int32),
        pltpu.VMEM((NUM_GRAPHS + 1, CW), jnp.float32),
        pltpu.SemaphoreType.DMA,
        pltpu.SemaphoreType.DMA,
    ],
)
def _sc_segsum(x_hbm, ids_hbm, out_hbm, xbuf, idbuf, acc, semx, semi):
    c = lax.axis_index("c")
    s = lax.axis_index("s")
    w = c * NS + s

    iota = lax.iota(jnp.int32, 16)
    cntv = jnp.where(iota == 0, 1.0, 0.0).astype(jnp.float32)
    cnt16v = jnp.where(iota == 0, 16.0, 0.0).astype(jnp.float32)
    zeros16 = jnp.zeros((16,), jnp.float32)

    def zero_body(i, carry):
        for g in range(CW // 16):
            acc[i, pl.ds(g * 16, 16)] = zeros16
        return carry

    lax.fori_loop(0, NUM_GRAPHS + 1, zero_body, 0, unroll=4)

    start = w * BASE_BLK + jnp.minimum(w, EXTRA)
    nblk = jnp.where(w < EXTRA, BASE_BLK + 1, BASE_BLK)

    def issue(i, par):
        blk = start + i
        pltpu.async_copy(
            x_hbm.at[pl.ds(blk * BLK, BLK), :],
            xbuf.at[pl.ds(par * BLK, BLK), :], semx)
        pltpu.async_copy(
            ids_hbm.at[pl.ds(blk * BLK, BLK)], idbuf.at[par, pl.ds(0, BLK)],
            semi)

    def drain(i, par):
        blk = start + i
        pltpu.make_async_copy(
            x_hbm.at[pl.ds(blk * BLK, BLK), :],
            xbuf.at[pl.ds(par * BLK, BLK), :], semx).wait()
        pltpu.make_async_copy(
            ids_hbm.at[pl.ds(blk * BLK, BLK)], idbuf.at[par, pl.ds(0, BLK)],
            semi).wait()

    for j in range(3):
        issue(j, j)

    def blk_body(i, carry):
        par = jnp.bitwise_and(i, 3)
        drain(i, par)

        @pl.when(i + 3 < nblk)
        def _():
            issue(i + 3, jnp.bitwise_and(i + 3, 3))

        pb = par * BLK

        def grp_body(gidx, gc):
            r0 = pb + gidx * 16
            idvec = idbuf[par, pl.ds(gidx * 16, 16)]
            seg0 = idvec[0]
            last = idvec[15]
            uniform = seg0 == last

            gs = [xbuf[r0, pl.ds(g * 16, 16)] for g in range(D_IN // 16)]
            for r in range(1, 16):
                for g in range(D_IN // 16):
                    gs[g] = gs[g] + xbuf[r0 + r, pl.ds(g * 16, 16)]

            @pl.when(uniform)
            def _():
                for g in range(D_IN // 16):
                    acc[seg0, pl.ds(g * 16, 16)] = (
                        acc[seg0, pl.ds(g * 16, 16)] + gs[g])
                acc[seg0, pl.ds(D_IN, 16)] = acc[seg0, pl.ds(D_IN, 16)] + cnt16v

            @pl.when(jnp.logical_not(uniform))
            def _():
                for r in range(16):
                    seg_r = idvec[r]
                    for g in range(D_IN // 16):
                        acc[seg_r, pl.ds(g * 16, 16)] = (
                            acc[seg_r, pl.ds(g * 16, 16)]
                            + xbuf[r0 + r, pl.ds(g * 16, 16)])
                    acc[seg_r, pl.ds(D_IN, 16)] = (
                        acc[seg_r, pl.ds(D_IN, 16)] + cntv)

            return gc

        return lax.fori_loop(0, BLK // 16, grp_body, carry)

    lax.fori_loop(0, nblk, blk_body, 0)

    pltpu.sync_copy(acc.at[pl.ds(0, NUM_GRAPHS), :], out_hbm.at[w])


def _tc_body(sacc_ref, w_in_ref, b_in_ref, w_pred_ref, b_pred_ref, out_ref):
    a = jnp.sum(sacc_ref[...], axis=0)
    sx = a[:, :D_IN]
    ext = a[:, D_IN:]
    row0 = lax.broadcasted_iota(jnp.int32, (16, HIDDEN), 0) == 0
    b16 = jnp.where(row0, jnp.broadcast_to(b_in_ref[...], (16, HIDDEN)), 0.0)
    pooled = jnp.dot(sx, w_in_ref[...], preferred_element_type=jnp.float32)
    pooled = pooled + jnp.dot(ext, b16, preferred_element_type=jnp.float32)
    out_ref[...] = (
        jnp.dot(pooled, w_pred_ref[...], preferred_element_type=jnp.float32)
        + b_pred_ref[...]
    )


def kernel(X, params, graph_segment_ids, W_in, b_in, W_pred, b_pred):
    del params
    ids32 = graph_segment_ids.astype(jnp.int32)
    sacc = _sc_segsum(X, ids32)
    out = pl.pallas_call(
        _tc_body,
        out_shape=jax.ShapeDtypeStruct((NUM_GRAPHS, OUT), jnp.float32),
    )(sacc, W_in, b_in.reshape(1, HIDDEN), W_pred, b_pred.reshape(1, OUT))
    return out
